# Initial kernel scaffold; baseline (speedup 1.0000x reference)
import jax
import jax.numpy as jnp
from jax.experimental import pallas as pl


def kernel(x, mask_token):
    raise NotImplementedError("write your pallas kernel here")


# speedup vs baseline: 1.2408x; 1.2408x over previous
import jax
import jax.numpy as jnp
from jax.experimental import pallas as pl
from jax.experimental.pallas import tpu as pltpu

_B, _R, _C, _D = 64, 32, 32, 768
_NROWS = _B * _R * _C
_NDROP = max(1, int(_R * _C * 0.25))
_BLK = 2048


def _flat_drop_ids():
    b_rand = jax.random.randint(jax.random.key(1), (_NDROP,), 0, _B)
    r_rand = jax.random.randint(jax.random.key(2), (_NDROP,), 0, _R)
    c_rand = jax.random.randint(jax.random.key(3), (_NDROP,), 0, _C)
    return (b_rand * _R + r_rand) * _C + c_rand


def _masked_copy_body(idx_ref, x_ref, tok_ref, o_ref):
    row0 = pl.program_id(0) * _BLK
    rows = jax.lax.broadcasted_iota(jnp.int32, (_BLK, 1), 0) + row0
    hit = jnp.any(rows == idx_ref[0, :][None, :], axis=-1, keepdims=True)
    o_ref[...] = jnp.where(hit, tok_ref[...], x_ref[...])


def kernel(x, mask_token):
    x2 = x.reshape(_NROWS, _D)
    tok = mask_token.reshape(1, _D)
    idx = _flat_drop_ids().reshape(1, _NDROP)
    out = pl.pallas_call(
        _masked_copy_body,
        grid=(_NROWS // _BLK,),
        in_specs=[
            pl.BlockSpec((1, _NDROP), lambda i: (0, 0)),
            pl.BlockSpec((_BLK, _D), lambda i: (i, 0)),
            pl.BlockSpec((1, _D), lambda i: (0, 0)),
        ],
        out_specs=pl.BlockSpec((_BLK, _D), lambda i: (i, 0)),
        out_shape=jax.ShapeDtypeStruct((_NROWS, _D), x.dtype),
        compiler_params=pltpu.CompilerParams(
            dimension_semantics=("arbitrary",),
        ),
    )(idx, x2, tok)
    return out.reshape(_B, _R, _C, _D)
